# baseline (speedup 1.0000x reference)
import functools
import math

import jax
import jax.numpy as jnp
from jax import lax
from jax.experimental import pallas as pl
from jax.experimental.pallas import tpu as pltpu

NUM_GRAPHS = 256
WG = 40


def _body(g0_ref, ovf_ref,
          h_ref, bat_ref, aux_ref,
          w1_ref, b1_ref, w2_ref, b2_ref,
          hp_ref, hpb_ref, dtab_ref,
          fw1a_ref, fw1b_ref, fw1c_ref, fb1_ref, fw2_ref, fb2_ref,
          gam_ref, bet_ref,
          out_ref,
          acc_num, acc_bh, acc_den, acc_cnt, acc_dep,
          *, num_heads, d_model, max_depth):
    step = pl.program_id(0)
    nsteps = pl.num_programs(0)
    f32 = jnp.float32

    @pl.when(step == 0)
    def _zero():
        acc_num[...] = jnp.zeros_like(acc_num)
        acc_bh[...] = jnp.zeros_like(acc_bh)
        acc_den[...] = jnp.zeros_like(acc_den)
        acc_cnt[...] = jnp.zeros_like(acc_cnt)
        acc_dep[...] = jnp.zeros_like(acc_dep)

    h = h_ref[...]
    batr = bat_ref[...].reshape(1, -1)
    aux = aux_ref[...]
    bsz = h.shape[0]
    bf16 = jnp.bfloat16

    hb16 = h.astype(bf16)
    t = jnp.tanh(lax.dot(hb16, w1_ref[...], preferred_element_type=f32)
                 + b1_ref[...])
    s = lax.dot(t, w2_ref[...], preferred_element_type=f32) + b2_ref[...]
    ex = jnp.exp(s)

    exb = ex.astype(bf16)
    mskb = aux[:, max_depth:max_depth + 1]
    rhs = jnp.concatenate(
        [hb16 * exb[:, i:i + 1] for i in range(num_heads)] + [hb16 * mskb],
        axis=1)
    g0 = pl.multiple_of(g0_ref[step], 8)
    ovf = ovf_ref[step]

    def _reduce(oneT, base):
        hgt = oneT.shape[0]
        m = lax.dot(oneT, rhs, preferred_element_type=f32)
        for i in range(num_heads):
            acc_num[pl.ds(base, hgt), d_model * i:d_model * (i + 1)] += (
                m[:, d_model * i:d_model * (i + 1)])
        acc_bh[pl.ds(base, hgt), :] += m[:, d_model * num_heads:]
        acc_den[pl.ds(base, hgt), :] += lax.dot(oneT, exb,
                                                preferred_element_type=f32)
        m2 = lax.dot(oneT, aux, preferred_element_type=f32)
        acc_dep[pl.ds(base, hgt), :] += m2[:, :max_depth]
        acc_cnt[pl.ds(base, hgt), :] += m2[:, max_depth:max_depth + 1]

    @pl.when(ovf == 0)
    def _window():
        wi = lax.broadcasted_iota(jnp.int32, (WG, bsz), 0)
        _reduce(((batr - g0) == wi).astype(bf16), g0)

    @pl.when(ovf != 0)
    def _full():
        gi = lax.broadcasted_iota(jnp.int32, (NUM_GRAPHS, bsz), 0)
        _reduce((batr == gi).astype(bf16), 0)

    @pl.when(step == nsteps - 1)
    def _epilogue():
        num = acc_num[...]
        den = acc_den[...]
        rd = jnp.where(den > 0.0, 1.0 / den, 0.0)
        hi = lax.broadcasted_iota(jnp.int32, (num_heads, num_heads * d_model), 0)
        ci = lax.broadcasted_iota(jnp.int32, (num_heads, num_heads * d_model), 1) // d_model
        expand = (hi == ci).astype(f32)
        pooled = num * lax.dot(rd, expand, preferred_element_type=f32)
        hg = lax.dot(pooled, hp_ref[...], preferred_element_type=f32) + hpb_ref[...]

        hb = acc_bh[...] / (acc_cnt[...] + 1e-08)

        cntd = acc_dep[...]
        dvals = lax.broadcasted_iota(jnp.int32, (NUM_GRAPHS, max_depth), 1).astype(f32)
        mdep = jnp.max(jnp.where(cntd > 0.0, dvals, 0.0), axis=1, keepdims=True)
        doh2 = (mdep == dvals).astype(f32)
        de = lax.dot(doh2, dtab_ref[...], preferred_element_type=f32)

        x = (lax.dot(hg, fw1a_ref[...], preferred_element_type=f32)
             + lax.dot(hb, fw1b_ref[...], preferred_element_type=f32)
             + lax.dot(de, fw1c_ref[...], preferred_element_type=f32)
             + fb1_ref[...])
        g = 0.5 * x * (1.0 + lax.erf(x * (1.0 / math.sqrt(2.0))))
        y = lax.dot(g, fw2_ref[...], preferred_element_type=f32) + fb2_ref[...]
        mu = jnp.mean(y, axis=1, keepdims=True)
        var = jnp.mean((y - mu) ** 2, axis=1, keepdims=True)
        out_ref[...] = (y - mu) * lax.rsqrt(var + 1e-05) * gam_ref[...] + bet_ref[...]


def kernel(h, batch, is_branch, depth, attn_W1, attn_b1, attn_W2, attn_b2,
           head_proj_W, head_proj_b, depth_table, fuse_W1, fuse_b1, fuse_W2,
           fuse_b2, ln_gamma, ln_beta):
    n, d_model = h.shape
    num_heads, _, d_head = attn_W1.shape
    max_depth = depth_table.shape[0]
    f32 = jnp.float32

    bsz = 4000 if n % 4000 == 0 else n
    grid = n // bsz

    w1 = jnp.transpose(attn_W1, (1, 0, 2)).reshape(
        d_model, num_heads * d_head).astype(jnp.bfloat16)
    b1 = attn_b1.reshape(1, num_heads * d_head)
    w2 = jnp.einsum('ik,ij->ikj', attn_W2[:, :, 0],
                    jnp.eye(num_heads, dtype=f32)).reshape(num_heads * d_head, num_heads)
    b2 = attn_b2.reshape(1, num_heads)

    bat2 = batch.reshape(n // bsz, 1, bsz)
    aux = jnp.concatenate(
        [(depth[:, None] == jnp.arange(max_depth, dtype=depth.dtype)[None, :]),
         is_branch[:, None]] + [jnp.zeros((n, 7), bool)],
        axis=1).astype(jnp.bfloat16)

    g_lo = batch[0::bsz]
    g_hi = batch[bsz - 1::bsz]
    g0 = jnp.minimum((g_lo // 8) * 8, NUM_GRAPHS - WG).astype(jnp.int32)
    ovf = (g_hi - g0 >= WG).astype(jnp.int32)

    fw1a = fuse_W1[:d_model]
    fw1b = fuse_W1[d_model:2 * d_model]
    fw1c = fuse_W1[2 * d_model:]

    row = lambda v: v.reshape(1, -1)

    body = functools.partial(_body, num_heads=num_heads, d_model=d_model,
                             max_depth=max_depth)
    blk = lambda shape: pl.BlockSpec(shape, lambda i, *_: (i, 0))
    whole = lambda a: pl.BlockSpec(a.shape, lambda i, *_: (0, 0))

    args = (h, bat2, aux, w1, b1, w2, b2,
            head_proj_W, row(head_proj_b), depth_table,
            fw1a, fw1b, fw1c, row(fuse_b1), fuse_W2, row(fuse_b2),
            row(ln_gamma), row(ln_beta))
    in_specs = [blk((bsz, d_model)),
                pl.BlockSpec((1, 1, bsz), lambda i, *_: (i, 0, 0)),
                blk((bsz, max_depth + 8))]
    in_specs += [whole(a) for a in args[3:]]

    return pl.pallas_call(
        body,
        grid_spec=pltpu.PrefetchScalarGridSpec(
            num_scalar_prefetch=2,
            grid=(grid,),
            in_specs=in_specs,
            out_specs=pl.BlockSpec((NUM_GRAPHS, d_model), lambda i, *_: (0, 0)),
            scratch_shapes=[
                pltpu.VMEM((NUM_GRAPHS, num_heads * d_model), f32),
                pltpu.VMEM((NUM_GRAPHS, d_model), f32),
                pltpu.VMEM((NUM_GRAPHS, num_heads), f32),
                pltpu.VMEM((NUM_GRAPHS, 1), f32),
                pltpu.VMEM((NUM_GRAPHS, max_depth), f32),
            ],
        ),
        out_shape=jax.ShapeDtypeStruct((NUM_GRAPHS, d_model), f32),
        compiler_params=pltpu.CompilerParams(
            dimension_semantics=("arbitrary",),
        ),
    )(g0, ovf, *args)


# speedup vs baseline: 1.0493x; 1.0493x over previous
import functools
import math

import jax
import jax.numpy as jnp
from jax import lax
from jax.experimental import pallas as pl
from jax.experimental.pallas import tpu as pltpu

NUM_GRAPHS = 256
WG = 40


def _body(g0_ref, ovf_ref,
          h_ref, bat_ref, aux_ref,
          w1_ref, b1_ref, w2_ref, b2_ref,
          hp_ref, hpb_ref, dtab_ref,
          fw1a_ref, fw1b_ref, fw1c_ref, fb1_ref, fw2_ref, fb2_ref,
          gam_ref, bet_ref,
          out_ref,
          acc_num, acc_bh, acc_den, acc_cnt, acc_dep,
          *, num_heads, d_model, max_depth):
    step = pl.program_id(0)
    nsteps = pl.num_programs(0)
    f32 = jnp.float32

    @pl.when(step == 0)
    def _zero():
        acc_num[...] = jnp.zeros_like(acc_num)
        acc_bh[...] = jnp.zeros_like(acc_bh)
        acc_den[...] = jnp.zeros_like(acc_den)
        acc_cnt[...] = jnp.zeros_like(acc_cnt)
        acc_dep[...] = jnp.zeros_like(acc_dep)

    h = h_ref[...]
    batr = bat_ref[...].reshape(1, -1)
    aux = aux_ref[...]
    bsz = h.shape[0]
    bf16 = jnp.bfloat16

    hb16 = h.astype(bf16)
    t = jnp.tanh(lax.dot(hb16, w1_ref[...], preferred_element_type=f32)
                 + b1_ref[...])
    s = lax.dot(t, w2_ref[...], preferred_element_type=f32) + b2_ref[...]
    ex = jnp.exp(s)

    exb = ex.astype(bf16)
    mskb = aux[:, max_depth:max_depth + 1]
    rhs = jnp.concatenate(
        [hb16 * exb[:, i:i + 1] for i in range(num_heads)] + [hb16 * mskb],
        axis=1)
    g0 = pl.multiple_of(g0_ref[step], 8)
    ovf = ovf_ref[step]

    def _reduce(oneT, base):
        hgt = oneT.shape[0]
        m = lax.dot(oneT, rhs, preferred_element_type=f32)
        for i in range(num_heads):
            acc_num[pl.ds(base, hgt), d_model * i:d_model * (i + 1)] += (
                m[:, d_model * i:d_model * (i + 1)])
        acc_bh[pl.ds(base, hgt), :] += m[:, d_model * num_heads:]
        acc_den[pl.ds(base, hgt), :] += lax.dot(oneT, exb,
                                                preferred_element_type=f32)
        m2 = lax.dot(oneT, aux, preferred_element_type=f32)
        acc_dep[pl.ds(base, hgt), :] += m2[:, :max_depth]
        acc_cnt[pl.ds(base, hgt), :] += m2[:, max_depth:max_depth + 1]

    @pl.when(ovf == 0)
    def _window():
        wi = lax.broadcasted_iota(jnp.int32, (WG, bsz), 0)
        _reduce(((batr - g0) == wi).astype(bf16), g0)

    @pl.when(ovf != 0)
    def _full():
        gi = lax.broadcasted_iota(jnp.int32, (NUM_GRAPHS, bsz), 0)
        _reduce((batr == gi).astype(bf16), 0)

    @pl.when(step == nsteps - 1)
    def _epilogue():
        num = acc_num[...]
        den = acc_den[...]
        rd = jnp.where(den > 0.0, 1.0 / den, 0.0)
        hi = lax.broadcasted_iota(jnp.int32, (num_heads, num_heads * d_model), 0)
        ci = lax.broadcasted_iota(jnp.int32, (num_heads, num_heads * d_model), 1) // d_model
        expand = (hi == ci).astype(f32)
        pooled = num * lax.dot(rd, expand, preferred_element_type=f32)
        hg = lax.dot(pooled, hp_ref[...], preferred_element_type=f32) + hpb_ref[...]

        hb = acc_bh[...] / (acc_cnt[...] + 1e-08)

        cntd = acc_dep[...]
        dvals = lax.broadcasted_iota(jnp.int32, (NUM_GRAPHS, max_depth), 1).astype(f32)
        mdep = jnp.max(jnp.where(cntd > 0.0, dvals, 0.0), axis=1, keepdims=True)
        doh2 = (mdep == dvals).astype(f32)
        de = lax.dot(doh2, dtab_ref[...], preferred_element_type=f32)

        x = (lax.dot(hg, fw1a_ref[...], preferred_element_type=f32)
             + lax.dot(hb, fw1b_ref[...], preferred_element_type=f32)
             + lax.dot(de, fw1c_ref[...], preferred_element_type=f32)
             + fb1_ref[...])
        g = 0.5 * x * (1.0 + lax.erf(x * (1.0 / math.sqrt(2.0))))
        y = lax.dot(g, fw2_ref[...], preferred_element_type=f32) + fb2_ref[...]
        mu = jnp.mean(y, axis=1, keepdims=True)
        var = jnp.mean((y - mu) ** 2, axis=1, keepdims=True)
        out_ref[...] = (y - mu) * lax.rsqrt(var + 1e-05) * gam_ref[...] + bet_ref[...]


def kernel(h, batch, is_branch, depth, attn_W1, attn_b1, attn_W2, attn_b2,
           head_proj_W, head_proj_b, depth_table, fuse_W1, fuse_b1, fuse_W2,
           fuse_b2, ln_gamma, ln_beta):
    n, d_model = h.shape
    num_heads, _, d_head = attn_W1.shape
    max_depth = depth_table.shape[0]
    f32 = jnp.float32

    bsz = 5000 if n % 5000 == 0 else n
    grid = n // bsz

    w1 = jnp.transpose(attn_W1, (1, 0, 2)).reshape(
        d_model, num_heads * d_head).astype(jnp.bfloat16)
    b1 = attn_b1.reshape(1, num_heads * d_head)
    w2 = jnp.einsum('ik,ij->ikj', attn_W2[:, :, 0],
                    jnp.eye(num_heads, dtype=f32)).reshape(num_heads * d_head, num_heads)
    b2 = attn_b2.reshape(1, num_heads)

    bat2 = batch.reshape(n // bsz, 1, bsz)
    aux = jnp.concatenate(
        [(depth[:, None] == jnp.arange(max_depth, dtype=depth.dtype)[None, :]),
         is_branch[:, None]] + [jnp.zeros((n, 7), bool)],
        axis=1).astype(jnp.bfloat16)

    g_lo = batch[0::bsz]
    g_hi = batch[bsz - 1::bsz]
    g0 = jnp.minimum((g_lo // 8) * 8, NUM_GRAPHS - WG).astype(jnp.int32)
    ovf = (g_hi - g0 >= WG).astype(jnp.int32)

    fw1a = fuse_W1[:d_model]
    fw1b = fuse_W1[d_model:2 * d_model]
    fw1c = fuse_W1[2 * d_model:]

    row = lambda v: v.reshape(1, -1)

    body = functools.partial(_body, num_heads=num_heads, d_model=d_model,
                             max_depth=max_depth)
    blk = lambda shape: pl.BlockSpec(shape, lambda i, *_: (i, 0))
    whole = lambda a: pl.BlockSpec(a.shape, lambda i, *_: (0, 0))

    args = (h, bat2, aux, w1, b1, w2, b2,
            head_proj_W, row(head_proj_b), depth_table,
            fw1a, fw1b, fw1c, row(fuse_b1), fuse_W2, row(fuse_b2),
            row(ln_gamma), row(ln_beta))
    in_specs = [blk((bsz, d_model)),
                pl.BlockSpec((1, 1, bsz), lambda i, *_: (i, 0, 0)),
                blk((bsz, max_depth + 8))]
    in_specs += [whole(a) for a in args[3:]]

    return pl.pallas_call(
        body,
        grid_spec=pltpu.PrefetchScalarGridSpec(
            num_scalar_prefetch=2,
            grid=(grid,),
            in_specs=in_specs,
            out_specs=pl.BlockSpec((NUM_GRAPHS, d_model), lambda i, *_: (0, 0)),
            scratch_shapes=[
                pltpu.VMEM((NUM_GRAPHS, num_heads * d_model), f32),
                pltpu.VMEM((NUM_GRAPHS, d_model), f32),
                pltpu.VMEM((NUM_GRAPHS, num_heads), f32),
                pltpu.VMEM((NUM_GRAPHS, 1), f32),
                pltpu.VMEM((NUM_GRAPHS, max_depth), f32),
            ],
        ),
        out_shape=jax.ShapeDtypeStruct((NUM_GRAPHS, d_model), f32),
        compiler_params=pltpu.CompilerParams(
            dimension_semantics=("arbitrary",),
        ),
    )(g0, ovf, *args)
